# Initial kernel scaffold; baseline (speedup 1.0000x reference)
import jax
import jax.numpy as jnp
from jax.experimental import pallas as pl


def kernel(x, edge_index, edge_type, edge_norm, node_norm, first_prev_graph_embeds, second_prev_graph_embeds, time_diff_tensor, W1, loop_w1, time_w1, W2, loop_w2, time_w2):
    raise NotImplementedError("write your pallas kernel here")


# speedup vs baseline: 27.2899x; 27.2899x over previous
import functools

import jax
import jax.numpy as jnp
from jax import lax
from jax.experimental import pallas as pl
from jax.experimental.pallas import tpu as pltpu
from jax.experimental.pallas import tpu_sc as plsc

N = 10000
E = 160000
D = 128
NB = 32
S = 4
NCORES = 2
NSUB = 16
NW = NCORES * NSUB
EPT = E // NW
C = 40
NCH = EPT // C
NPAD = 10240
RPT = NPAD // NSUB
ZR = 128
INV_T = 0.1
NVR = D // 16


def _edge_pass_body(h_hbm, src_hbm, dst_hbm, et_hbm, en_hbm, wt_hbm, out_hbm,
                    src_v, dst_v, et_v, en_v, hs_v, w_v, msg_v, z_v, agg,
                    sem_h, sem_w):
    c = lax.axis_index("c")
    s = lax.axis_index("s")
    wid = c * NSUB + s
    ebase = wid * EPT
    rbase = s * RPT

    def zrow(j, carry):
        zero = jnp.zeros((16,), jnp.float32)
        for v in range(NVR):
            z_v[j, pl.ds(16 * v, 16)] = zero
        return carry
    lax.fori_loop(0, ZR, zrow, 0)
    for t in range(RPT // ZR):
        pltpu.sync_copy(z_v, agg.at[pl.ds(rbase + t * ZR, ZR)])
    plsc.subcore_barrier()

    pat0 = (lax.iota(jnp.int32, 16) // S) * S

    def chunk(g, carry):
        eb = ebase + g * C
        pltpu.sync_copy(src_hbm.at[pl.ds(eb, C)], src_v)
        pltpu.sync_copy(dst_hbm.at[pl.ds(eb, C)], dst_v)
        pltpu.sync_copy(et_hbm.at[pl.ds(eb, C)], et_v)
        pltpu.sync_copy(en_hbm.at[pl.ds(eb, C)], en_v)
        cp_h = pltpu.async_copy(h_hbm.at[src_v], hs_v, sem_h)
        cp_w = pltpu.async_copy(wt_hbm.at[et_v], w_v, sem_w)
        cp_h.wait()
        cp_w.wait()

        def edge(le, ecarry):
            lev = jnp.full((16,), le, jnp.int32)
            en = plsc.load_gather(en_v, [lev])
            for v in range(NVR):
                acc = None
                for i in range(S):
                    hsel = plsc.load_gather(hs_v, [lev, pat0 + (16 * v + i)])
                    wv = w_v[le, pl.ds(i * D + 16 * v, 16)]
                    term = hsel * wv
                    acc = term if acc is None else acc + term
                msg_v[le, pl.ds(16 * v, 16)] = acc * en
            return ecarry
        lax.fori_loop(0, C, edge, 0)

        pltpu.sync_copy(msg_v, agg.at[dst_v], add=True)
        return carry
    lax.fori_loop(0, NCH, chunk, 0)

    plsc.subcore_barrier()
    pltpu.sync_copy(agg.at[pl.ds(rbase, RPT)], out_hbm.at[c, pl.ds(rbase, RPT)])


_edge_pass = functools.partial(
    pl.kernel,
    mesh=plsc.VectorSubcoreMesh(core_axis_name="c", subcore_axis_name="s"),
    out_type=jax.ShapeDtypeStruct((NCORES, NPAD, D), jnp.float32),
    scratch_types=[
        pltpu.VMEM((C,), jnp.int32),
        pltpu.VMEM((C,), jnp.int32),
        pltpu.VMEM((C,), jnp.int32),
        pltpu.VMEM((C,), jnp.float32),
        pltpu.VMEM((C, D), jnp.float32),
        pltpu.VMEM((C, S * D), jnp.float32),
        pltpu.VMEM((C, D), jnp.float32),
        pltpu.VMEM((ZR, D), jnp.float32),
        pltpu.VMEM_SHARED((NPAD, D), jnp.float32),
        pltpu.SemaphoreType.DMA,
        pltpu.SemaphoreType.DMA,
    ],
    compiler_params=pltpu.CompilerParams(needs_layout_passes=False),
)(_edge_pass_body)


def _epi_body(act, agg_ref, nn_ref, td_ref, prev_ref, h_ref, tw_ref, lw_ref,
              out_ref):
    agg = agg_ref[0] + agg_ref[1]
    nn = nn_ref[...]
    td = td_ref[...]
    t = jnp.dot(prev_ref[...], tw_ref[...], preferred_element_type=jnp.float32)
    l = jnp.dot(h_ref[...], lw_ref[...], preferred_element_type=jnp.float32)
    r = agg * nn + t * jnp.exp(td * (-INV_T)) + l
    if act:
        r = jnp.maximum(r, 0.0)
    out_ref[...] = r


def _epilogue(agg, nn, td, prev, h, tw, lw, act):
    R = 2000
    grid = (N // R,)
    return pl.pallas_call(
        functools.partial(_epi_body, act),
        grid=grid,
        in_specs=[
            pl.BlockSpec((NCORES, R, D), lambda i: (0, i, 0)),
            pl.BlockSpec((R, 1), lambda i: (i, 0)),
            pl.BlockSpec((R, 1), lambda i: (i, 0)),
            pl.BlockSpec((R, D), lambda i: (i, 0)),
            pl.BlockSpec((R, D), lambda i: (i, 0)),
            pl.BlockSpec((D, D), lambda i: (0, 0)),
            pl.BlockSpec((D, D), lambda i: (0, 0)),
        ],
        out_specs=pl.BlockSpec((R, D), lambda i: (i, 0)),
        out_shape=jax.ShapeDtypeStruct((N, D), jnp.float32),
    )(agg, nn, td, prev, h, tw, lw)


def kernel(x, edge_index, edge_type, edge_norm, node_norm,
           first_prev_graph_embeds, second_prev_graph_embeds,
           time_diff_tensor, W1, loop_w1, time_w1, W2, loop_w2, time_w2):
    src = edge_index[0]
    dst = edge_index[1]
    en = edge_norm[:, 0]
    nn = node_norm
    td = time_diff_tensor
    w1t = W1.reshape(-1, NB, S, S).transpose(0, 2, 1, 3).reshape(-1, NB * S * S)
    w2t = W2.reshape(-1, NB, S, S).transpose(0, 2, 1, 3).reshape(-1, NB * S * S)

    agg1 = _edge_pass(x, src, dst, edge_type, en, w1t)
    h1 = _epilogue(agg1, nn, td, first_prev_graph_embeds, x, time_w1, loop_w1,
                   act=False)
    agg2 = _edge_pass(h1, src, dst, edge_type, en, w2t)
    h2 = _epilogue(agg2, nn, td, second_prev_graph_embeds, h1, time_w2,
                   loop_w2, act=True)
    return (h1, h2)


# speedup vs baseline: 73.8612x; 2.7065x over previous
import functools

import jax
import jax.numpy as jnp
from jax import lax
from jax.experimental import pallas as pl
from jax.experimental.pallas import tpu as pltpu
from jax.experimental.pallas import tpu_sc as plsc

N = 10000
E = 160000
D = 128
NB = 32
S = 4
NCORES = 2
NSUB = 16
NW = NCORES * NSUB
EPT = E // NW
C = 25
NCH = EPT // C
DEPTH = 2
RPT = N // NSUB
ZR = 25
INV_T = 0.1
NVR = D // 16
SDMASK = 0x3FFF
ETMASK = 0x7FF


def _cast(x):
    return x.astype(jnp.int32) if x.dtype != jnp.int32 else x


_GDN = lax.GatherDimensionNumbers(
    offset_dims=(), collapsed_slice_dims=(0,), start_index_map=(0,))


def _vperm(x, idx):
    return lax.gather(x, idx[:, None], _GDN, slice_sizes=(1,),
                      mode=lax.GatherScatterMode.PROMISE_IN_BOUNDS)


def _edge_pass_body(h_hbm, pkr_hbm, wt_hbm, out_hbm,
                    pk_v, et_i, hs_v, w_v, msg_v, agg, *sems):
    sem_h = sems[:DEPTH]
    sem_w = sems[DEPTH:2 * DEPTH]
    sem_s = sems[2 * DEPTH:]
    c = lax.axis_index("c")
    s = lax.axis_index("s")
    wid = c * NSUB + s
    rbase = s * RPT

    def zrow(j, carry):
        zero = jnp.zeros((16,), jnp.float32)
        for v in range(NVR):
            msg_v[0][j, pl.ds(16 * v, 16)] = zero
        return carry
    lax.fori_loop(0, ZR, zrow, 0)
    for t in range(RPT // ZR):
        pltpu.async_copy(msg_v[0].at[pl.ds(0, ZR)],
                         agg.at[pl.ds(rbase + t * ZR, ZR)], sem_s[0])
    for t in range(RPT // ZR):
        pltpu.make_async_copy(msg_v[0].at[pl.ds(0, ZR)],
                              agg.at[pl.ds(rbase + t * ZR, ZR)],
                              sem_s[0]).wait()

    pltpu.sync_copy(pkr_hbm.at[wid], pk_v)
    plsc.subcore_barrier()

    iota = lax.iota(jnp.int32, 16)
    pat_a = (iota // S) * 2
    pats = (pat_a, pat_a + 1)

    def prep(g, r):
        for k in (0, C - 16):
            ev = pk_v[2 * NCH + g, pl.ds(k, 16)]
            et_i[r][pl.ds(k, 16)] = ev & ETMASK

    def gathers(g, r):
        return (
            pltpu.make_async_copy(h_hbm.at[pk_v.at[g]], hs_v[r], sem_h[r]),
            pltpu.make_async_copy(wt_hbm.at[et_i[r]], w_v[r], sem_w[r]),
        )

    def compute(g, r):
        def edge(le, ecarry):
            lev = jnp.full((16,), le, jnp.int32)
            en_i = plsc.load_gather(pk_v, [jnp.full((16,), 2 * NCH + g,
                                                    jnp.int32), lev])
            en = plsc.bitcast(en_i & jnp.int32(~ETMASK), jnp.float32)
            msk = msg_v[r]
            for u in range(NVR // 2):
                hab = hs_v[r][le, pl.ds(32 * u, 32)]
                eab = plsc.unpack(hab, format=plsc.PackFormat.INTERLEAVED)
                acc_a = None
                acc_b = None
                for i in range(S):
                    wab = w_v[r][le, pl.ds(32 * (S * i + u), 32)]
                    wa, wb = plsc.unpack(
                        wab, format=plsc.PackFormat.INTERLEAVED)
                    hsrc = eab[i % 2]
                    p = pats[i // 2]
                    ta = _vperm(hsrc, p) * wa
                    tb = _vperm(hsrc, p + 8) * wb
                    acc_a = ta if acc_a is None else acc_a + ta
                    acc_b = tb if acc_b is None else acc_b + tb
                msk[le, pl.ds(32 * u, 16)] = acc_a * en
                msk[le, pl.ds(32 * u + 16, 16)] = acc_b * en
            return ecarry
        lax.fori_loop(0, C, edge, 0)

    def scatter_start(g, r):
        pltpu.async_copy(msg_v[r], agg.at[pk_v.at[NCH + g]], sem_s[r],
                         add=True)

    def scatter_wait(g, r):
        pltpu.make_async_copy(msg_v[r], agg.at[pk_v.at[NCH + g]],
                              sem_s[r]).wait()

    for r in range(DEPTH):
        prep(jnp.int32(r), r)
        for cp in gathers(jnp.int32(r), r):
            cp.start()

    def piter(t, carry):
        g0 = DEPTH * t
        for r in range(DEPTH):
            g = g0 + r
            for cp in gathers(g, r):
                cp.wait()

            @pl.when(t > 0)
            def _():
                scatter_wait(g - DEPTH, r)
            compute(g, r)
            scatter_start(g, r)
            gn = jnp.minimum(g + DEPTH, NCH - 1)
            prep(gn, r)
            for cp in gathers(gn, r):
                cp.start()
        return carry
    lax.fori_loop(0, NCH // DEPTH, piter, 0)

    for r in range(DEPTH):
        for cp in gathers(jnp.int32(NCH - 1), r):
            cp.wait()
        scatter_wait(jnp.int32(NCH - DEPTH + r), r)

    plsc.subcore_barrier()
    pltpu.sync_copy(agg.at[pl.ds(rbase, RPT)],
                    out_hbm.at[c, pl.ds(rbase, RPT)])


_edge_pass = functools.partial(
    pl.kernel,
    mesh=plsc.VectorSubcoreMesh(core_axis_name="c", subcore_axis_name="s"),
    out_type=jax.ShapeDtypeStruct((NCORES, N, D), jnp.float32),
    scratch_types=(
        [pltpu.VMEM((3 * NCH, C), jnp.int32)]
        + [[pltpu.VMEM((C,), jnp.int32) for _ in range(DEPTH)]]
        + [[pltpu.VMEM((C, D), jnp.bfloat16) for _ in range(DEPTH)]]
        + [[pltpu.VMEM((C, S * D), jnp.bfloat16) for _ in range(DEPTH)]]
        + [[pltpu.VMEM((C, D), jnp.float32) for _ in range(DEPTH)]]
        + [pltpu.VMEM_SHARED((N, D), jnp.float32)]
        + [pltpu.SemaphoreType.DMA] * (3 * DEPTH)
    ),
    compiler_params=pltpu.CompilerParams(needs_layout_passes=False,
                                         use_tc_tiling_on_sc=False),
)(_edge_pass_body)


def _epi_body(act, emit_bf, agg_ref, nn_ref, td_ref, prev_ref, h_ref, tw_ref,
              lw_ref, out_ref, bf_ref=None):
    agg = agg_ref[0] + agg_ref[1]
    nn = nn_ref[...]
    td = td_ref[...]
    t = jnp.dot(prev_ref[...], tw_ref[...], preferred_element_type=jnp.float32)
    l = jnp.dot(h_ref[...], lw_ref[...], preferred_element_type=jnp.float32)
    r = agg * nn + t * jnp.exp(td * (-INV_T)) + l
    if act:
        r = jnp.maximum(r, 0.0)
    out_ref[...] = r
    if emit_bf:
        bf_ref[...] = r.astype(jnp.bfloat16)


def _epilogue(agg, nn, td, prev, h, tw, lw, act, emit_bf):
    R = 2000
    grid = (N // R,)
    out_shape = [jax.ShapeDtypeStruct((N, D), jnp.float32)]
    out_specs = [pl.BlockSpec((R, D), lambda i: (i, 0))]
    if emit_bf:
        out_shape.append(jax.ShapeDtypeStruct((N, D), jnp.bfloat16))
        out_specs.append(pl.BlockSpec((R, D), lambda i: (i, 0)))
    res = pl.pallas_call(
        functools.partial(_epi_body, act, emit_bf),
        grid=grid,
        in_specs=[
            pl.BlockSpec((NCORES, R, D), lambda i: (0, i, 0)),
            pl.BlockSpec((R, 1), lambda i: (i, 0)),
            pl.BlockSpec((R, 1), lambda i: (i, 0)),
            pl.BlockSpec((R, D), lambda i: (i, 0)),
            pl.BlockSpec((R, D), lambda i: (i, 0)),
            pl.BlockSpec((D, D), lambda i: (0, 0)),
            pl.BlockSpec((D, D), lambda i: (0, 0)),
        ],
        out_specs=out_specs,
        out_shape=out_shape,
    )(agg, nn, td, prev, h, tw, lw)
    return res if emit_bf else res[0]


def _wprep(w):
    wt = w.reshape(-1, NB, S, S).transpose(0, 2, 1, 3)
    wt = wt.reshape(-1, S, S, 2, 16).transpose(0, 1, 2, 4, 3)
    return wt.reshape(-1, NB * S * S).astype(jnp.bfloat16)


def kernel(x, edge_index, edge_type, edge_norm, node_norm,
           first_prev_graph_embeds, second_prev_graph_embeds,
           time_diff_tensor, W1, loop_w1, time_w1, W2, loop_w2, time_w2):
    src = _cast(edge_index[0])
    dst = _cast(edge_index[1])
    et = _cast(edge_type)
    en_bits = lax.bitcast_convert_type(edge_norm[:, 0], jnp.int32)
    nn = node_norm
    td = time_diff_tensor
    w1t = _wprep(W1)
    w2t = _wprep(W2)
    pk_ee = (en_bits & ~ETMASK) | et
    pkr = jnp.concatenate(
        [f.reshape(NW, NCH, C) for f in (src, dst, pk_ee)], axis=1)
    x_bf = x.astype(jnp.bfloat16)

    agg1 = _edge_pass(x_bf, pkr, w1t)
    h1, h1_bf = _epilogue(agg1, nn, td, first_prev_graph_embeds, x, time_w1,
                          loop_w1, act=False, emit_bf=True)
    agg2 = _edge_pass(h1_bf, pkr, w2t)
    h2 = _epilogue(agg2, nn, td, second_prev_graph_embeds, h1, time_w2,
                   loop_w2, act=True, emit_bf=False)
    return (h1, h2)
